# Initial kernel scaffold; baseline (speedup 1.0000x reference)
import jax
import jax.numpy as jnp
from jax.experimental import pallas as pl


def kernel(output, target, norm):
    raise NotImplementedError("write your pallas kernel here")


# speedup vs baseline: 1.1157x; 1.1157x over previous
import functools
import math

import jax
import jax.numpy as jnp
from jax import lax
from jax.experimental import pallas as pl
from jax.experimental.pallas import tpu as pltpu
from jax.experimental.pallas import tpu_sc as plsc

B = 1024
V = 100000
N_FLAT = B * V
SMOOTH = 0.1 / (V - 2)
CONF = 0.9
Z_COL = V - 100
A_CONST = SMOOTH * math.log(SMOOTH) * (V - 2) + CONF * math.log(CONF)
SLOGS = SMOOTH * math.log(SMOOTH)

NW = 32
ROWS_PER_W = B // NW
ELEMS_PER_W = ROWS_PER_W * V
CHUNK = 40000
NCHUNK = ELEMS_PER_W // CHUNK
NVEC = CHUNK // 16
UNROLL = 20
NACC = 5


def _sum_chunk(buf, accs):
    def body(i, accs):
        a = list(accs)
        base = i * (UNROLL * 16)
        for u in range(UNROLL):
            v = buf[pl.ds(base + u * 16, 16)]
            a[u % NACC] = a[u % NACC] + v
        return tuple(a)
    return lax.fori_loop(0, NVEC // UNROLL, body, accs)


def _sc_body(flat_hbm, tgt_hbm, out_hbm, buf0, buf1, tgt_v, idx_v, gat_v,
             out_v, sem0, sem1, semg):
    wid = lax.axis_index("s") * 2 + lax.axis_index("c")
    base = wid * ELEMS_PER_W

    pltpu.sync_copy(tgt_hbm.at[pl.ds(wid * ROWS_PER_W, ROWS_PER_W)], tgt_v)
    iota = lax.iota(jnp.int32, 16)
    row0 = wid * ROWS_PER_W
    for k in range(2):
        t16 = tgt_v[pl.ds(k * 16, 16)]
        t16c = jnp.minimum(jnp.maximum(t16, 0), V - 1)
        rows = (row0 + k * 16 + iota) * V
        idx_v[pl.ds(k * 16, 16)] = rows + t16c
        idx_v[pl.ds(32 + k * 16, 16)] = rows + Z_COL
    gather = pltpu.async_copy(flat_hbm.at[idx_v], gat_v, semg)

    pltpu.async_copy(flat_hbm.at[pl.ds(base, CHUNK)], buf0, sem0)
    zero = jnp.zeros((16,), jnp.float32)
    accs0 = (zero,) * NACC

    def chunk_body(k, accs):
        j = 2 * k
        off = base + j * CHUNK
        pltpu.make_async_copy(flat_hbm.at[pl.ds(base, CHUNK)], buf0, sem0).wait()
        pltpu.async_copy(flat_hbm.at[pl.ds(off + CHUNK, CHUNK)], buf1, sem1)
        accs = _sum_chunk(buf0, accs)

        @pl.when(k < NCHUNK // 2 - 1)
        def _():
            pltpu.async_copy(flat_hbm.at[pl.ds(off + 2 * CHUNK, CHUNK)], buf0,
                             sem0)

        pltpu.make_async_copy(flat_hbm.at[pl.ds(base, CHUNK)], buf1, sem1).wait()
        accs = _sum_chunk(buf1, accs)
        return accs

    accs = lax.fori_loop(0, NCHUNK // 2, chunk_body, accs0)
    total_vec = accs[0]
    for a in accs[1:]:
        total_vec = total_vec + a

    gather.wait()
    vec = jnp.float32(-SMOOTH) * total_vec
    for k in range(2):
        t16 = tgt_v[pl.ds(k * 16, 16)]
        tv = gat_v[pl.ds(k * 16, 16)]
        zv = gat_v[pl.ds(32 + k * 16, 16)]
        isz = jnp.where(t16 == Z_COL, jnp.float32(1.0), jnp.float32(0.0))
        vec = vec + (jnp.float32(SMOOTH) * zv + jnp.float32(SMOOTH - CONF) * tv
                     + isz * (jnp.float32(SLOGS) - jnp.float32(SMOOTH) * zv))
    vec = vec + jnp.where(iota == 0, jnp.float32(ROWS_PER_W * A_CONST),
                          jnp.float32(0.0))
    out_v[...] = vec
    pltpu.sync_copy(out_v, out_hbm.at[wid])


@jax.jit
def _sc_loss(flat, target):
    mesh = plsc.VectorSubcoreMesh(core_axis_name="c", subcore_axis_name="s")
    f = pl.kernel(
        _sc_body,
        out_type=jax.ShapeDtypeStruct((NW, 16), jnp.float32),
        mesh=mesh,
        scratch_types=[
            pltpu.VMEM((CHUNK,), jnp.float32),
            pltpu.VMEM((CHUNK,), jnp.float32),
            pltpu.VMEM((ROWS_PER_W,), jnp.int32),
            pltpu.VMEM((64,), jnp.int32),
            pltpu.VMEM((64,), jnp.float32),
            pltpu.VMEM((16,), jnp.float32),
            pltpu.SemaphoreType.DMA,
            pltpu.SemaphoreType.DMA,
            pltpu.SemaphoreType.DMA,
        ],
    )
    return f(flat, target)


def kernel(output, target, norm):
    partials = _sc_loss(output.reshape(N_FLAT), target)
    return jnp.sum(partials) / jnp.asarray(norm).astype(jnp.float32)


# speedup vs baseline: 1.7056x; 1.5288x over previous
import functools
import math

import jax
import jax.numpy as jnp
from jax import lax
from jax.experimental import pallas as pl
from jax.experimental.pallas import tpu as pltpu
from jax.experimental.pallas import tpu_sc as plsc

B = 1024
V = 100000
SMOOTH = 0.1 / (V - 2)
CONF = 0.9
Z_COL = V - 100
A_CONST = SMOOTH * math.log(SMOOTH) * (V - 2) + CONF * math.log(CONF)
SLOGS = SMOOTH * math.log(SMOOTH)

R_SC = 512
R_TC = B - R_SC

NW = 32
RG_PER_W = R_SC // (NW * 8)
ROWS_PER_W = RG_PER_W * 8
MAIN_W = 2560
NMAIN = 39
TAIL_C0 = NMAIN * MAIN_W
TAIL_W = 128
EDGE_C0 = TAIL_C0 + TAIL_W
EDGE_W = 32
Z_LOCAL = Z_COL - TAIL_C0
UNROLL = 10
NACC = 10
NVEC_MAIN = MAIN_W // 16

TC_RB = 256
TC_CB = 512
TC_NR = R_TC // TC_RB
TC_NC = (V + TC_CB - 1) // TC_CB


def _sum_row(buf, rr, nvec, accs):
    if nvec <= UNROLL:
        a = list(accs)
        for u in range(nvec):
            a[u % NACC] = a[u % NACC] + buf[rr, pl.ds(u * 16, 16)]
        return tuple(a)

    def body(i, accs):
        a = list(accs)
        base = i * (UNROLL * 16)
        for u in range(UNROLL):
            a[u] = a[u] + buf[rr, pl.ds(base + u * 16, 16)]
        return tuple(a)
    return lax.fori_loop(0, nvec // UNROLL, body, accs)


def _sum_block(buf, nvec, accs):
    for rr in range(8):
        accs = _sum_row(buf, rr, nvec, accs)
    return accs


def _sc_body(out_hbm, tgt_hbm, edge_hbm, res_hbm, buf0, buf1, tailb, edgeb,
             tgt_v, out_v, sem0, sem1, semt, seme):
    wid = lax.axis_index("s") * 2 + lax.axis_index("c")
    row0 = wid * ROWS_PER_W
    pltpu.sync_copy(tgt_hbm.at[pl.ds(row0, ROWS_PER_W)],
                    tgt_v.at[pl.ds(0, ROWS_PER_W)])
    pltpu.async_copy(edge_hbm, edgeb, seme)
    iota = lax.iota(jnp.int32, 16)
    lane0 = jnp.where(iota == 0, jnp.float32(1.0), jnp.float32(0.0))
    zero = jnp.zeros((16,), jnp.float32)

    def get_t(rg, rr):
        return tgt_v[pl.ds((rg // 2) * 16, 16)][(rg % 2) * 8 + rr]

    def sparse_tv(buf, rg, rr, c0, width, svec):
        t = get_t(rg, rr)
        tl = t - c0
        in_f = jnp.where((tl >= 0) & (tl < width),
                         jnp.float32(SMOOTH - CONF), jnp.float32(0.0))
        tlc = jnp.minimum(jnp.maximum(tl, 0), width - 1)
        span = buf[rr, pl.ds(pl.multiple_of(tlc & ~15, 16), 16)]
        lane_m = jnp.where(iota == (tlc & 15), in_f, jnp.float32(0.0))
        return svec + lane_m * span

    def sparse_z(rg, rr, svec):
        t = get_t(rg, rr)
        zspan = tailb[rr, pl.ds((Z_LOCAL // 16) * 16, 16)]
        isz = jnp.where(t == Z_COL, jnp.float32(1.0), jnp.float32(0.0))
        zterm = (jnp.float32(SMOOTH) * zspan
                 + isz * (jnp.float32(SLOGS) - jnp.float32(SMOOTH) * zspan))
        return svec + jnp.where(iota == (Z_LOCAL % 16), zterm,
                                jnp.float32(0.0))

    carry = (zero,) * NACC + (zero,)
    for rg in range(RG_PER_W):
        r8 = row0 + rg * 8

        def dma_main(j, buf, sem):
            return pltpu.async_copy(
                out_hbm.at[pl.ds(r8, 8), pl.ds(j * MAIN_W, MAIN_W)], buf, sem)

        dma_main(0, buf0, sem0)

        def pair_body(k, carry, rg=rg, r8=r8, dma_main=dma_main):
            accs, svec = carry[:NACC], carry[NACC]
            j = 2 * k
            pltpu.make_async_copy(
                out_hbm.at[pl.ds(0, 8), pl.ds(0, MAIN_W)], buf0, sem0).wait()
            dma_main(j + 1, buf1, sem1)
            accs = _sum_block(buf0, NVEC_MAIN, accs)
            c0 = j * MAIN_W
            for rr in range(8):
                svec = sparse_tv(buf0, rg, rr, c0, MAIN_W, svec)
            dma_main(j + 2, buf0, sem0)
            pltpu.make_async_copy(
                out_hbm.at[pl.ds(0, 8), pl.ds(0, MAIN_W)], buf1, sem1).wait()
            accs = _sum_block(buf1, NVEC_MAIN, accs)
            c1 = c0 + MAIN_W
            for rr in range(8):
                svec = sparse_tv(buf1, rg, rr, c1, MAIN_W, svec)
            return accs + (svec,)

        carry = lax.fori_loop(0, (NMAIN - 1) // 2, pair_body, carry)

        pltpu.async_copy(
            out_hbm.at[pl.ds(r8, 8), pl.ds(TAIL_C0, TAIL_W)], tailb, semt)
        accs, svec = carry[:NACC], carry[NACC]
        pltpu.make_async_copy(
            out_hbm.at[pl.ds(0, 8), pl.ds(0, MAIN_W)], buf0, sem0).wait()
        accs = _sum_block(buf0, NVEC_MAIN, accs)
        for rr in range(8):
            svec = sparse_tv(buf0, rg, rr, (NMAIN - 1) * MAIN_W, MAIN_W, svec)
        pltpu.make_async_copy(
            out_hbm.at[pl.ds(0, 8), pl.ds(0, TAIL_W)], tailb, semt).wait()
        accs = _sum_block(tailb, TAIL_W // 16, accs)
        for rr in range(8):
            svec = sparse_tv(tailb, rg, rr, TAIL_C0, TAIL_W, svec)
            svec = sparse_z(rg, rr, svec)
        carry = accs + (svec,)

    accs, svec = carry[:NACC], carry[NACC]
    pltpu.make_async_copy(edge_hbm, edgeb, seme).wait()
    ngr = (ROWS_PER_W + 15) // 16
    for g in range(ngr):
        nlanes = min(16, ROWS_PER_W - g * 16)
        col0 = pl.multiple_of(row0 + g * 16, 16)
        t16 = tgt_v[pl.ds(g * 16, 16)]
        lanes_ok = jnp.where(iota < nlanes, jnp.float32(1.0),
                             jnp.float32(0.0))
        esum = zero
        for i in range(EDGE_W):
            v = edgeb[i, pl.ds(col0, 16)]
            esum = esum + lanes_ok * v
            m1 = jnp.where(iota < nlanes, jnp.float32(SMOOTH - CONF),
                           jnp.float32(0.0))
            m = jnp.where(t16 == (EDGE_C0 + i), m1, jnp.float32(0.0))
            svec = svec + m * v
        svec = svec + jnp.float32(-SMOOTH) * esum
    total_vec = accs[0]
    for a in accs[1:]:
        total_vec = total_vec + a

    vec = (jnp.float32(-SMOOTH) * total_vec + svec
           + lane0 * jnp.float32(ROWS_PER_W * A_CONST))
    out_v[...] = vec
    pltpu.sync_copy(out_v, res_hbm.at[pl.ds(wid * 16, 16)])


def _tc_body(x_ref, t_ref, o_ref):
    i = pl.program_id(0)
    j = pl.program_id(1)
    x = x_ref[...]
    t = t_ref[...]
    colmat = j * TC_CB + lax.broadcasted_iota(jnp.int32, (TC_RB, TC_CB), 1)
    m_t = colmat == t
    oob_or_z = (colmat >= V) | (colmat == Z_COL)
    w = jnp.where(m_t, jnp.float32(-CONF),
                  jnp.where(oob_or_z, jnp.float32(0.0),
                            jnp.float32(-SMOOTH)))
    xs = jnp.where(colmat >= V, jnp.float32(0.0), x)
    contrib = xs * w
    tz = jnp.where((t == Z_COL) & (colmat == 0), jnp.float32(SLOGS),
                   jnp.float32(0.0))
    contrib = contrib + tz
    part = contrib.reshape(TC_RB // 8, 8, TC_CB // 128, 128).sum(axis=(0, 2))

    @pl.when((i == 0) & (j == 0))
    def _():
        o_ref[...] = jnp.zeros_like(o_ref)

    o_ref[...] += part


@jax.jit
def _loss_parts(output, target):
    mesh = plsc.VectorSubcoreMesh(core_axis_name="c", subcore_axis_name="s")
    edge_t = output[:, EDGE_C0:].T
    sc = pl.kernel(
        _sc_body,
        out_type=jax.ShapeDtypeStruct((NW * 16,), jnp.float32),
        mesh=mesh,
        scratch_types=[
            pltpu.VMEM((8, MAIN_W), jnp.float32),
            pltpu.VMEM((8, MAIN_W), jnp.float32),
            pltpu.VMEM((8, TAIL_W), jnp.float32),
            pltpu.VMEM((EDGE_W, B), jnp.float32),
            pltpu.VMEM((max(ROWS_PER_W, 16),), jnp.int32),
            pltpu.VMEM((16,), jnp.float32),
            pltpu.SemaphoreType.DMA,
            pltpu.SemaphoreType.DMA,
            pltpu.SemaphoreType.DMA,
            pltpu.SemaphoreType.DMA,
        ],
    )(output, target, edge_t)

    tc = pl.pallas_call(
        _tc_body,
        out_shape=jax.ShapeDtypeStruct((8, 128), jnp.float32),
        grid=(TC_NR, TC_NC),
        in_specs=[
            pl.BlockSpec((TC_RB, TC_CB), lambda i, j: (i + R_SC // TC_RB, j)),
            pl.BlockSpec((TC_RB, 1), lambda i, j: (i + R_SC // TC_RB, 0)),
        ],
        out_specs=pl.BlockSpec((8, 128), lambda i, j: (0, 0)),
        compiler_params=pltpu.CompilerParams(
            dimension_semantics=("arbitrary", "arbitrary")),
    )(output, target[:, None])
    return sc, tc


def kernel(output, target, norm):
    sc, tc = _loss_parts(output, target)
    total = jnp.sum(sc) + jnp.sum(tc) + jnp.float32(R_TC * A_CONST)
    return total / jnp.asarray(norm).astype(jnp.float32)


# speedup vs baseline: 2.1208x; 1.2434x over previous
import functools
import math

import jax
import jax.numpy as jnp
from jax import lax
from jax.experimental import pallas as pl
from jax.experimental.pallas import tpu as pltpu
from jax.experimental.pallas import tpu_sc as plsc

B = 1024
V = 100000
SMOOTH = 0.1 / (V - 2)
CONF = 0.9
Z_COL = V - 100
A_CONST = SMOOTH * math.log(SMOOTH) * (V - 2) + CONF * math.log(CONF)
SLOGS = SMOOTH * math.log(SMOOTH)

R_SC = 768
R_TC = B - R_SC

NW = 32
RG_PER_W = R_SC // (NW * 8)
ROWS_PER_W = RG_PER_W * 8
MAIN_W = 2560
NMAIN = 39
TAIL_C0 = NMAIN * MAIN_W
TAIL_W = 128
EDGE_C0 = TAIL_C0 + TAIL_W
EDGE_W = V - EDGE_C0
Z_LOCAL = Z_COL - TAIL_C0
UNROLL = 10
NACC = 10
NVEC_MAIN = MAIN_W // 16

TC_RB = 256
TC_CB = 512
TC_NR = R_TC // TC_RB
TC_NC = (V + TC_CB - 1) // TC_CB


def _sum_row(buf, rr, nvec, accs):
    if nvec <= UNROLL:
        a = list(accs)
        for u in range(nvec):
            a[u % NACC] = a[u % NACC] + buf[rr, pl.ds(u * 16, 16)]
        return tuple(a)

    def body(i, accs):
        a = list(accs)
        base = i * (UNROLL * 16)
        for u in range(UNROLL):
            a[u] = a[u] + buf[rr, pl.ds(base + u * 16, 16)]
        return tuple(a)
    return lax.fori_loop(0, nvec // UNROLL, body, accs)


def _sum_block(buf, nvec, accs):
    for rr in range(8):
        accs = _sum_row(buf, rr, nvec, accs)
    return accs


def _sc_body(out_hbm, tgt_hbm, res_hbm, buf0, buf1, tailb, tgt_v, out_v,
             sem0, sem1, semt):
    wid = lax.axis_index("s") * 2 + lax.axis_index("c")
    row0 = wid * ROWS_PER_W
    pltpu.sync_copy(tgt_hbm.at[pl.ds(row0, ROWS_PER_W)],
                    tgt_v.at[pl.ds(0, ROWS_PER_W)])
    iota = lax.iota(jnp.int32, 16)
    lane0 = jnp.where(iota == 0, jnp.float32(1.0), jnp.float32(0.0))
    zero = jnp.zeros((16,), jnp.float32)

    def get_t(rg, rr):
        return tgt_v[pl.ds((rg // 2) * 16, 16)][(rg % 2) * 8 + rr]

    def sparse_tv(buf, rg, rr, c0, width, svec):
        t = get_t(rg, rr)
        tl = t - c0
        in_f = jnp.where((tl >= 0) & (tl < width),
                         jnp.float32(SMOOTH - CONF), jnp.float32(0.0))
        tlc = jnp.minimum(jnp.maximum(tl, 0), width - 1)
        span = buf[rr, pl.ds(pl.multiple_of(tlc & ~15, 16), 16)]
        lane_m = jnp.where(iota == (tlc & 15), in_f, jnp.float32(0.0))
        return svec + lane_m * span

    def sparse_z(rg, rr, svec):
        t = get_t(rg, rr)
        zspan = tailb[rr, pl.ds((Z_LOCAL // 16) * 16, 16)]
        isz = jnp.where(t == Z_COL, jnp.float32(1.0), jnp.float32(0.0))
        zterm = (jnp.float32(SMOOTH) * zspan
                 + isz * (jnp.float32(SLOGS) - jnp.float32(SMOOTH) * zspan))
        return svec + jnp.where(iota == (Z_LOCAL % 16), zterm,
                                jnp.float32(0.0))

    carry = (zero,) * NACC + (zero,)
    for rg in range(RG_PER_W):
        r8 = row0 + rg * 8

        def dma_main(j, buf, sem):
            return pltpu.async_copy(
                out_hbm.at[pl.ds(r8, 8), pl.ds(j * MAIN_W, MAIN_W)], buf, sem)

        dma_main(0, buf0, sem0)

        def pair_body(k, carry, rg=rg, r8=r8, dma_main=dma_main):
            accs, svec = carry[:NACC], carry[NACC]
            j = 2 * k
            pltpu.make_async_copy(
                out_hbm.at[pl.ds(0, 8), pl.ds(0, MAIN_W)], buf0, sem0).wait()
            dma_main(j + 1, buf1, sem1)
            accs = _sum_block(buf0, NVEC_MAIN, accs)
            c0 = j * MAIN_W
            for rr in range(8):
                svec = sparse_tv(buf0, rg, rr, c0, MAIN_W, svec)
            dma_main(j + 2, buf0, sem0)
            pltpu.make_async_copy(
                out_hbm.at[pl.ds(0, 8), pl.ds(0, MAIN_W)], buf1, sem1).wait()
            accs = _sum_block(buf1, NVEC_MAIN, accs)
            c1 = c0 + MAIN_W
            for rr in range(8):
                svec = sparse_tv(buf1, rg, rr, c1, MAIN_W, svec)
            return accs + (svec,)

        carry = lax.fori_loop(0, (NMAIN - 1) // 2, pair_body, carry)

        pltpu.async_copy(
            out_hbm.at[pl.ds(r8, 8), pl.ds(TAIL_C0, TAIL_W)], tailb, semt)
        accs, svec = carry[:NACC], carry[NACC]
        pltpu.make_async_copy(
            out_hbm.at[pl.ds(0, 8), pl.ds(0, MAIN_W)], buf0, sem0).wait()
        accs = _sum_block(buf0, NVEC_MAIN, accs)
        for rr in range(8):
            svec = sparse_tv(buf0, rg, rr, (NMAIN - 1) * MAIN_W, MAIN_W, svec)
        pltpu.make_async_copy(
            out_hbm.at[pl.ds(0, 8), pl.ds(0, TAIL_W)], tailb, semt).wait()
        accs = _sum_block(tailb, TAIL_W // 16, accs)
        for rr in range(8):
            svec = sparse_tv(tailb, rg, rr, TAIL_C0, TAIL_W, svec)
            svec = sparse_z(rg, rr, svec)
        carry = accs + (svec,)

    accs, svec = carry[:NACC], carry[NACC]
    total_vec = accs[0]
    for a in accs[1:]:
        total_vec = total_vec + a

    vec = (jnp.float32(-SMOOTH) * total_vec + svec
           + lane0 * jnp.float32(ROWS_PER_W * A_CONST))
    out_v[...] = vec
    pltpu.sync_copy(out_v, res_hbm.at[pl.ds(wid * 16, 16)])


def _tc_body(x_ref, t_ref, o_ref):
    i = pl.program_id(0)
    j = pl.program_id(1)
    x = x_ref[...]
    t = t_ref[...]
    colmat = j * TC_CB + lax.broadcasted_iota(jnp.int32, (TC_RB, TC_CB), 1)
    m_t = colmat == t
    oob_or_z = (colmat >= V) | (colmat == Z_COL)
    w = jnp.where(m_t, jnp.float32(-CONF),
                  jnp.where(oob_or_z, jnp.float32(0.0),
                            jnp.float32(-SMOOTH)))
    xs = jnp.where(colmat >= V, jnp.float32(0.0), x)
    contrib = xs * w
    tz = jnp.where((t == Z_COL) & (colmat == 0), jnp.float32(SLOGS),
                   jnp.float32(0.0))
    contrib = contrib + tz
    part = contrib.reshape(TC_RB // 8, 8, TC_CB // 128, 128).sum(axis=(0, 2))

    @pl.when((i == 0) & (j == 0))
    def _():
        o_ref[...] = jnp.zeros_like(o_ref)

    o_ref[...] += part


def _tc_edge_body(x_ref, t_ref, o_ref):
    x = x_ref[...]
    t = t_ref[...]
    c0 = (TC_NC - 1) * TC_CB
    colmat = c0 + lax.broadcasted_iota(jnp.int32, (R_SC, TC_CB), 1)
    live = (colmat >= EDGE_C0) & (colmat < V)
    w = jnp.where(live & (colmat == t), jnp.float32(-CONF),
                  jnp.where(live, jnp.float32(-SMOOTH), jnp.float32(0.0)))
    xs = jnp.where(colmat >= V, jnp.float32(0.0), x)
    part = (xs * w).reshape(R_SC // 8, 8, TC_CB // 128, 128).sum(axis=(0, 2))
    o_ref[...] = part


@jax.jit
def _loss_parts(output, target):
    mesh = plsc.VectorSubcoreMesh(core_axis_name="c", subcore_axis_name="s")
    sc = pl.kernel(
        _sc_body,
        out_type=jax.ShapeDtypeStruct((NW * 16,), jnp.float32),
        mesh=mesh,
        scratch_types=[
            pltpu.VMEM((8, MAIN_W), jnp.float32),
            pltpu.VMEM((8, MAIN_W), jnp.float32),
            pltpu.VMEM((8, TAIL_W), jnp.float32),
            pltpu.VMEM((32,), jnp.int32),
            pltpu.VMEM((16,), jnp.float32),
            pltpu.SemaphoreType.DMA,
            pltpu.SemaphoreType.DMA,
            pltpu.SemaphoreType.DMA,
        ],
    )(output, target)

    t2d = target[:, None]
    tc = pl.pallas_call(
        _tc_body,
        out_shape=jax.ShapeDtypeStruct((8, 128), jnp.float32),
        grid=(TC_NR, TC_NC),
        in_specs=[
            pl.BlockSpec((TC_RB, TC_CB), lambda i, j: (i + R_SC // TC_RB, j)),
            pl.BlockSpec((TC_RB, 1), lambda i, j: (i + R_SC // TC_RB, 0)),
        ],
        out_specs=pl.BlockSpec((8, 128), lambda i, j: (0, 0)),
        compiler_params=pltpu.CompilerParams(
            dimension_semantics=("arbitrary", "arbitrary")),
    )(output, t2d)

    tce = pl.pallas_call(
        _tc_edge_body,
        out_shape=jax.ShapeDtypeStruct((8, 128), jnp.float32),
        grid=(1,),
        in_specs=[
            pl.BlockSpec((R_SC, TC_CB), lambda i: (0, TC_NC - 1)),
            pl.BlockSpec((R_SC, 1), lambda i: (0, 0)),
        ],
        out_specs=pl.BlockSpec((8, 128), lambda i: (0, 0)),
    )(output, t2d)
    return sc, tc, tce


def kernel(output, target, norm):
    sc, tc, tce = _loss_parts(output, target)
    total = (jnp.sum(sc) + jnp.sum(tc) + jnp.sum(tce)
             + jnp.float32(R_TC * A_CONST))
    return total / jnp.asarray(norm).astype(jnp.float32)


# speedup vs baseline: 7.1000x; 3.3479x over previous
import functools
import math

import jax
import jax.numpy as jnp
from jax import lax
from jax.experimental import pallas as pl
from jax.experimental.pallas import tpu as pltpu
from jax.experimental.pallas import tpu_sc as plsc

B = 1024
V = 100000
SMOOTH = 0.1 / (V - 2)
CONF = 0.9
Z_COL = V - 100
A_CONST = SMOOTH * math.log(SMOOTH) * (V - 2) + CONF * math.log(CONF)
SLOGS = SMOOTH * math.log(SMOOTH)

NW = 32
CR = 32
RPW = 1568
V_SC = NW * RPW
NCH = RPW // CR
BPW = B // NW
UNROLL = 8
NACC = 8
NVEC_ROW = B // 16

TC_RB = 512
TC_R0 = V_SC // TC_RB
TC_NR = (V - V_SC + TC_RB - 1) // TC_RB


def _sc_body(outT_hbm, tgt_hbm, res_hbm, buf0, buf1, tbuf, zbuf, tgt_v,
             out_v, sem0, sem1, semg):
    wid = lax.axis_index("s") * 2 + lax.axis_index("c")
    v0 = wid * RPW
    b0 = wid * BPW
    pltpu.sync_copy(tgt_hbm.at[pl.ds(b0, BPW)], tgt_v)
    iota = lax.iota(jnp.int32, 16)
    lane0 = jnp.where(iota == 0, jnp.float32(1.0), jnp.float32(0.0))
    zero = jnp.zeros((16,), jnp.float32)

    def get_t(i):
        return tgt_v[pl.ds((i // 16) * 16, 16)][i % 16]

    bcol = pl.multiple_of((b0 & ~127), 128)
    pltpu.async_copy(
        outT_hbm.at[pl.ds((Z_COL // 8) * 8, 8), pl.ds(bcol, 128)], zbuf, semg)

    def dma_chunk(j, buf, sem):
        return pltpu.async_copy(
            outT_hbm.at[pl.ds(v0 + j * CR, CR), pl.ds(0, B)], buf, sem)

    def sum_chunk(buf, accs):
        def row_body(rr, accs):
            def body(ii, accs):
                a = list(accs)
                base = ii * (UNROLL * 16)
                for u in range(UNROLL):
                    a[u] = a[u] + buf[rr, pl.ds(base + u * 16, 16)]
                return tuple(a)
            return lax.fori_loop(0, NVEC_ROW // UNROLL, body, accs)
        return lax.fori_loop(0, CR, row_body, accs)

    dma_chunk(0, buf0, sem0)
    accs = (zero,) * NACC

    def pair_body(k, accs):
        j = 2 * k
        pltpu.make_async_copy(
            outT_hbm.at[pl.ds(0, CR), pl.ds(0, B)], buf0, sem0).wait()
        dma_chunk(j + 1, buf1, sem1)
        accs = sum_chunk(buf0, accs)
        dma_chunk(j + 2, buf0, sem0)
        pltpu.make_async_copy(
            outT_hbm.at[pl.ds(0, CR), pl.ds(0, B)], buf1, sem1).wait()
        accs = sum_chunk(buf1, accs)
        return accs

    accs = lax.fori_loop(0, (NCH - 1) // 2, pair_body, accs)
    pltpu.make_async_copy(
        outT_hbm.at[pl.ds(0, CR), pl.ds(0, B)], buf0, sem0).wait()
    accs = sum_chunk(buf0, accs)

    pltpu.make_async_copy(
        outT_hbm.at[pl.ds(0, 8), pl.ds(0, 128)], zbuf, semg).wait()

    svec = zero
    span0 = pl.multiple_of((b0 & 127) & ~15, 16)
    for wave in range(BPW // 8):
        for k in range(8):
            i = wave * 8 + k
            t = get_t(i)
            tr8 = pl.multiple_of(
                (jnp.minimum(jnp.maximum(t, 0), V - 1)) & ~7, 8)
            pltpu.async_copy(
                outT_hbm.at[pl.ds(tr8, 8), pl.ds(bcol, 128)],
                tbuf.at[k], semg)
        for k in range(8):
            pltpu.make_async_copy(
                outT_hbm.at[pl.ds(0, 8), pl.ds(0, 128)], tbuf.at[k],
                semg).wait()
        for k in range(8):
            i = wave * 8 + k
            t = get_t(i)
            lane = i & 15
            sp = span0 + (i // 16) * 16
            zspan = zbuf[Z_COL & 7, pl.ds(sp, 16)]
            isz = jnp.where(t == Z_COL, jnp.float32(1.0), jnp.float32(0.0))
            zterm = (jnp.float32(SMOOTH) * zspan
                     + isz * (jnp.float32(SLOGS)
                              - jnp.float32(SMOOTH) * zspan))
            svec = svec + jnp.where(iota == lane, zterm, jnp.float32(0.0))
            tv = zero
            for r8 in range(8):
                m = jnp.where((t & 7) == r8, jnp.float32(SMOOTH - CONF),
                              jnp.float32(0.0))
                tv = tv + m * tbuf[k, r8, pl.ds(sp, 16)]
            svec = svec + jnp.where(iota == lane, tv, jnp.float32(0.0))

    total_vec = accs[0]
    for a in accs[1:]:
        total_vec = total_vec + a
    vec = jnp.float32(-SMOOTH) * total_vec + svec
    out_v[...] = vec
    pltpu.sync_copy(out_v, res_hbm.at[pl.ds(wid * 16, 16)])


def _tc_body(x_ref, o_ref):
    i = pl.program_id(0)
    x = x_ref[...]
    r0 = (TC_R0 + i) * TC_RB
    rowmat = r0 + lax.broadcasted_iota(jnp.int32, (TC_RB, B), 0)
    xs = jnp.where(rowmat >= V, jnp.float32(0.0), x)
    part = xs.reshape(TC_RB // 8, 8, B // 128, 128).sum(axis=(0, 2))

    @pl.when(i == 0)
    def _():
        o_ref[...] = jnp.zeros_like(o_ref)

    o_ref[...] += part


@jax.jit
def _loss_parts(output, target):
    out_t = output.T
    mesh = plsc.VectorSubcoreMesh(core_axis_name="c", subcore_axis_name="s")
    sc = pl.kernel(
        _sc_body,
        out_type=jax.ShapeDtypeStruct((NW * 16,), jnp.float32),
        mesh=mesh,
        scratch_types=[
            pltpu.VMEM((CR, B), jnp.float32),
            pltpu.VMEM((CR, B), jnp.float32),
            pltpu.VMEM((8, 8, 128), jnp.float32),
            pltpu.VMEM((8, 128), jnp.float32),
            pltpu.VMEM((BPW,), jnp.int32),
            pltpu.VMEM((16,), jnp.float32),
            pltpu.SemaphoreType.DMA,
            pltpu.SemaphoreType.DMA,
            pltpu.SemaphoreType.DMA,
        ],
    )(out_t, target)

    tc = pl.pallas_call(
        _tc_body,
        out_shape=jax.ShapeDtypeStruct((8, 128), jnp.float32),
        grid=(TC_NR,),
        in_specs=[
            pl.BlockSpec((TC_RB, B), lambda i: (i + TC_R0, 0)),
        ],
        out_specs=pl.BlockSpec((8, 128), lambda i: (0, 0)),
        compiler_params=pltpu.CompilerParams(
            dimension_semantics=("arbitrary",)),
    )(out_t)
    return sc, tc


def kernel(output, target, norm):
    sc, tc = _loss_parts(output, target)
    total = (jnp.sum(sc) + jnp.float32(-SMOOTH) * jnp.sum(tc)
             + jnp.float32(B * A_CONST))
    return total / jnp.asarray(norm).astype(jnp.float32)


# speedup vs baseline: 7.3777x; 1.0391x over previous
import functools
import math

import jax
import jax.numpy as jnp
from jax import lax
from jax.experimental import pallas as pl
from jax.experimental.pallas import tpu as pltpu
from jax.experimental.pallas import tpu_sc as plsc

B = 1024
V = 100000
SMOOTH = 0.1 / (V - 2)
CONF = 0.9
Z_COL = V - 100
A_CONST = SMOOTH * math.log(SMOOTH) * (V - 2) + CONF * math.log(CONF)
SLOGS = SMOOTH * math.log(SMOOTH)

NW = 32
CR = 32
RPW = 1696
V_SC = NW * RPW
NCH = RPW // CR
BPW = B // NW
UNROLL = 8
NACC = 8
NVEC_ROW = B // 16

TC_RB = 1024
TC_R0 = V_SC // TC_RB
TC_NR = (V - V_SC + TC_RB - 1) // TC_RB


def _sc_body(outT_hbm, tgt_hbm, res_hbm, buf0, buf1, tbuf, zbuf, tgt_v,
             out_v, sem0, sem1, semg):
    wid = lax.axis_index("s") * 2 + lax.axis_index("c")
    v0 = wid * RPW
    b0 = wid * BPW
    pltpu.sync_copy(tgt_hbm.at[pl.ds(b0, BPW)], tgt_v)
    iota = lax.iota(jnp.int32, 16)
    lane0 = jnp.where(iota == 0, jnp.float32(1.0), jnp.float32(0.0))
    zero = jnp.zeros((16,), jnp.float32)

    def get_t(i):
        return tgt_v[pl.ds((i // 16) * 16, 16)][i % 16]

    bcol = pl.multiple_of((b0 & ~127), 128)
    pltpu.async_copy(
        outT_hbm.at[pl.ds((Z_COL // 8) * 8, 8), pl.ds(bcol, 128)], zbuf, semg)

    def dma_chunk(j, buf, sem):
        return pltpu.async_copy(
            outT_hbm.at[pl.ds(v0 + j * CR, CR), pl.ds(0, B)], buf, sem)

    def sum_chunk(buf, accs):
        def row_body(rr, accs):
            def body(ii, accs):
                a = list(accs)
                base = ii * (UNROLL * 16)
                for u in range(UNROLL):
                    a[u] = a[u] + buf[rr, pl.ds(base + u * 16, 16)]
                return tuple(a)
            return lax.fori_loop(0, NVEC_ROW // UNROLL, body, accs)
        return lax.fori_loop(0, CR, row_body, accs)

    dma_chunk(0, buf0, sem0)
    accs = (zero,) * NACC

    def pair_body(k, accs):
        j = 2 * k
        pltpu.make_async_copy(
            outT_hbm.at[pl.ds(0, CR), pl.ds(0, B)], buf0, sem0).wait()
        dma_chunk(j + 1, buf1, sem1)
        accs = sum_chunk(buf0, accs)
        dma_chunk(j + 2, buf0, sem0)
        pltpu.make_async_copy(
            outT_hbm.at[pl.ds(0, CR), pl.ds(0, B)], buf1, sem1).wait()
        accs = sum_chunk(buf1, accs)
        return accs

    accs = lax.fori_loop(0, (NCH - 1) // 2, pair_body, accs)
    pltpu.make_async_copy(
        outT_hbm.at[pl.ds(0, CR), pl.ds(0, B)], buf0, sem0).wait()
    accs = sum_chunk(buf0, accs)

    pltpu.make_async_copy(
        outT_hbm.at[pl.ds(0, 8), pl.ds(0, 128)], zbuf, semg).wait()

    svec = zero
    span0 = pl.multiple_of((b0 & 127) & ~15, 16)
    for wave in range(BPW // 8):
        for k in range(8):
            i = wave * 8 + k
            t = get_t(i)
            tr8 = pl.multiple_of(
                (jnp.minimum(jnp.maximum(t, 0), V - 1)) & ~7, 8)
            pltpu.async_copy(
                outT_hbm.at[pl.ds(tr8, 8), pl.ds(bcol, 128)],
                tbuf.at[k], semg)
        for k in range(8):
            pltpu.make_async_copy(
                outT_hbm.at[pl.ds(0, 8), pl.ds(0, 128)], tbuf.at[k],
                semg).wait()
        for k in range(8):
            i = wave * 8 + k
            t = get_t(i)
            lane = i & 15
            sp = span0 + (i // 16) * 16
            zspan = zbuf[Z_COL & 7, pl.ds(sp, 16)]
            isz = jnp.where(t == Z_COL, jnp.float32(1.0), jnp.float32(0.0))
            zterm = (jnp.float32(SMOOTH) * zspan
                     + isz * (jnp.float32(SLOGS)
                              - jnp.float32(SMOOTH) * zspan))
            svec = svec + jnp.where(iota == lane, zterm, jnp.float32(0.0))
            tv = zero
            for r8 in range(8):
                m = jnp.where((t & 7) == r8, jnp.float32(SMOOTH - CONF),
                              jnp.float32(0.0))
                tv = tv + m * tbuf[k, r8, pl.ds(sp, 16)]
            svec = svec + jnp.where(iota == lane, tv, jnp.float32(0.0))

    total_vec = accs[0]
    for a in accs[1:]:
        total_vec = total_vec + a
    vec = jnp.float32(-SMOOTH) * total_vec + svec
    out_v[...] = vec
    pltpu.sync_copy(out_v, res_hbm.at[pl.ds(wid * 16, 16)])


def _tc_body(x_ref, o_ref):
    i = pl.program_id(0)
    x = x_ref[...]

    @pl.when(i == 0)
    def _():
        o_ref[...] = jnp.zeros_like(o_ref)

    @pl.when(i < TC_NR - 1)
    def _():
        o_ref[...] += x.reshape(TC_RB // 8, 8, B // 128, 128).sum(axis=(0, 2))

    @pl.when(i == TC_NR - 1)
    def _():
        r0 = (TC_R0 + i) * TC_RB
        rowmat = r0 + lax.broadcasted_iota(jnp.int32, (TC_RB, B), 0)
        xs = jnp.where(rowmat >= V, jnp.float32(0.0), x)
        o_ref[...] += xs.reshape(TC_RB // 8, 8, B // 128, 128).sum(axis=(0, 2))


@jax.jit
def _loss_parts(output, target):
    out_t = output.T
    mesh = plsc.VectorSubcoreMesh(core_axis_name="c", subcore_axis_name="s")
    sc = pl.kernel(
        _sc_body,
        out_type=jax.ShapeDtypeStruct((NW * 16,), jnp.float32),
        mesh=mesh,
        scratch_types=[
            pltpu.VMEM((CR, B), jnp.float32),
            pltpu.VMEM((CR, B), jnp.float32),
            pltpu.VMEM((8, 8, 128), jnp.float32),
            pltpu.VMEM((8, 128), jnp.float32),
            pltpu.VMEM((BPW,), jnp.int32),
            pltpu.VMEM((16,), jnp.float32),
            pltpu.SemaphoreType.DMA,
            pltpu.SemaphoreType.DMA,
            pltpu.SemaphoreType.DMA,
        ],
    )(out_t, target)

    tc = pl.pallas_call(
        _tc_body,
        out_shape=jax.ShapeDtypeStruct((8, 128), jnp.float32),
        grid=(TC_NR,),
        in_specs=[
            pl.BlockSpec((TC_RB, B), lambda i: (i + TC_R0, 0)),
        ],
        out_specs=pl.BlockSpec((8, 128), lambda i: (0, 0)),
        compiler_params=pltpu.CompilerParams(
            dimension_semantics=("arbitrary",)),
    )(out_t)
    return sc, tc


def kernel(output, target, norm):
    sc, tc = _loss_parts(output, target)
    total = (jnp.sum(sc) + jnp.float32(-SMOOTH) * jnp.sum(tc)
             + jnp.float32(B * A_CONST))
    return total / jnp.asarray(norm).astype(jnp.float32)


# speedup vs baseline: 7.4894x; 1.0151x over previous
import functools
import math

import jax
import jax.numpy as jnp
from jax import lax
from jax.experimental import pallas as pl
from jax.experimental.pallas import tpu as pltpu
from jax.experimental.pallas import tpu_sc as plsc

B = 1024
V = 100000
SMOOTH = 0.1 / (V - 2)
CONF = 0.9
Z_COL = V - 100
A_CONST = SMOOTH * math.log(SMOOTH) * (V - 2) + CONF * math.log(CONF)
SLOGS = SMOOTH * math.log(SMOOTH)

NW = 32
CR = 32
RPW = 1568
V_SC = NW * RPW
NCH = RPW // CR
BPW = B // NW
UNROLL = 8
NACC = 8
NVEC_ROW = B // 16

TC_RB = 1024
TC_R0 = V_SC // TC_RB
TC_NR = (V - V_SC + TC_RB - 1) // TC_RB


def _sc_body(outT_hbm, tgt_hbm, res_hbm, buf0, buf1, tbuf, zbuf, tgt_v,
             out_v, sem0, sem1, semg):
    wid = lax.axis_index("s") * 2 + lax.axis_index("c")
    v0 = wid * RPW
    b0 = wid * BPW
    pltpu.sync_copy(tgt_hbm.at[pl.ds(b0, BPW)], tgt_v)
    iota = lax.iota(jnp.int32, 16)
    lane0 = jnp.where(iota == 0, jnp.float32(1.0), jnp.float32(0.0))
    zero = jnp.zeros((16,), jnp.float32)

    def get_t(i):
        return tgt_v[pl.ds((i // 16) * 16, 16)][i % 16]

    bcol = pl.multiple_of((b0 & ~127), 128)
    pltpu.async_copy(
        outT_hbm.at[pl.ds((Z_COL // 8) * 8, 8), pl.ds(bcol, 128)], zbuf, semg)

    def dma_chunk(j, buf, sem):
        return pltpu.async_copy(
            outT_hbm.at[pl.ds(v0 + j * CR, CR), pl.ds(0, B)], buf, sem)

    def sum_chunk(buf, accs):
        def row_body(rr, accs):
            def body(ii, accs):
                a = list(accs)
                base = ii * (UNROLL * 16)
                for u in range(UNROLL):
                    a[u] = a[u] + buf[rr, pl.ds(base + u * 16, 16)]
                return tuple(a)
            return lax.fori_loop(0, NVEC_ROW // UNROLL, body, accs)
        return lax.fori_loop(0, CR, row_body, accs)

    dma_chunk(0, buf0, sem0)
    accs = (zero,) * NACC

    def pair_body(k, accs):
        j = 2 * k
        pltpu.make_async_copy(
            outT_hbm.at[pl.ds(0, CR), pl.ds(0, B)], buf0, sem0).wait()
        dma_chunk(j + 1, buf1, sem1)
        accs = sum_chunk(buf0, accs)
        dma_chunk(j + 2, buf0, sem0)
        pltpu.make_async_copy(
            outT_hbm.at[pl.ds(0, CR), pl.ds(0, B)], buf1, sem1).wait()
        accs = sum_chunk(buf1, accs)
        return accs

    accs = lax.fori_loop(0, (NCH - 1) // 2, pair_body, accs)
    pltpu.make_async_copy(
        outT_hbm.at[pl.ds(0, CR), pl.ds(0, B)], buf0, sem0).wait()
    accs = sum_chunk(buf0, accs)

    pltpu.make_async_copy(
        outT_hbm.at[pl.ds(0, 8), pl.ds(0, 128)], zbuf, semg).wait()

    svec = zero
    span0 = pl.multiple_of((b0 & 127) & ~15, 16)
    for wave in range(BPW // 8):
        for k in range(8):
            i = wave * 8 + k
            t = get_t(i)
            tr8 = pl.multiple_of(
                (jnp.minimum(jnp.maximum(t, 0), V - 1)) & ~7, 8)
            pltpu.async_copy(
                outT_hbm.at[pl.ds(tr8, 8), pl.ds(bcol, 128)],
                tbuf.at[k], semg)
        for k in range(8):
            pltpu.make_async_copy(
                outT_hbm.at[pl.ds(0, 8), pl.ds(0, 128)], tbuf.at[k],
                semg).wait()
        for k in range(8):
            i = wave * 8 + k
            t = get_t(i)
            lane = i & 15
            sp = span0 + (i // 16) * 16
            zspan = zbuf[Z_COL & 7, pl.ds(sp, 16)]
            isz = jnp.where(t == Z_COL, jnp.float32(1.0), jnp.float32(0.0))
            zterm = (jnp.float32(SMOOTH) * zspan
                     + isz * (jnp.float32(SLOGS)
                              - jnp.float32(SMOOTH) * zspan))
            svec = svec + jnp.where(iota == lane, zterm, jnp.float32(0.0))
            tv = zero
            for r8 in range(8):
                m = jnp.where((t & 7) == r8, jnp.float32(SMOOTH - CONF),
                              jnp.float32(0.0))
                tv = tv + m * tbuf[k, r8, pl.ds(sp, 16)]
            svec = svec + jnp.where(iota == lane, tv, jnp.float32(0.0))

    total_vec = accs[0]
    for a in accs[1:]:
        total_vec = total_vec + a
    vec = jnp.float32(-SMOOTH) * total_vec + svec
    out_v[...] = vec
    pltpu.sync_copy(out_v, res_hbm.at[pl.ds(wid * 16, 16)])


def _tc_body(x_ref, o_ref):
    i = pl.program_id(0)
    x = x_ref[...]

    @pl.when(i == 0)
    def _():
        o_ref[...] = jnp.zeros_like(o_ref)

    @pl.when(i < TC_NR - 1)
    def _():
        o_ref[...] += x.reshape(TC_RB // 8, 8, B // 128, 128).sum(axis=(0, 2))

    @pl.when(i == TC_NR - 1)
    def _():
        r0 = (TC_R0 + i) * TC_RB
        rowmat = r0 + lax.broadcasted_iota(jnp.int32, (TC_RB, B), 0)
        xs = jnp.where(rowmat >= V, jnp.float32(0.0), x)
        o_ref[...] += xs.reshape(TC_RB // 8, 8, B // 128, 128).sum(axis=(0, 2))


@jax.jit
def _loss_parts(output, target):
    out_t = output.T
    mesh = plsc.VectorSubcoreMesh(core_axis_name="c", subcore_axis_name="s")
    sc = pl.kernel(
        _sc_body,
        out_type=jax.ShapeDtypeStruct((NW * 16,), jnp.float32),
        mesh=mesh,
        scratch_types=[
            pltpu.VMEM((CR, B), jnp.float32),
            pltpu.VMEM((CR, B), jnp.float32),
            pltpu.VMEM((8, 8, 128), jnp.float32),
            pltpu.VMEM((8, 128), jnp.float32),
            pltpu.VMEM((BPW,), jnp.int32),
            pltpu.VMEM((16,), jnp.float32),
            pltpu.SemaphoreType.DMA,
            pltpu.SemaphoreType.DMA,
            pltpu.SemaphoreType.DMA,
        ],
    )(out_t, target)

    tc = pl.pallas_call(
        _tc_body,
        out_shape=jax.ShapeDtypeStruct((8, 128), jnp.float32),
        grid=(TC_NR,),
        in_specs=[
            pl.BlockSpec((TC_RB, B), lambda i: (i + TC_R0, 0)),
        ],
        out_specs=pl.BlockSpec((8, 128), lambda i: (0, 0)),
        compiler_params=pltpu.CompilerParams(
            dimension_semantics=("arbitrary",)),
    )(out_t)
    return sc, tc


def kernel(output, target, norm):
    sc, tc = _loss_parts(output, target)
    total = (jnp.sum(sc) + jnp.float32(-SMOOTH) * jnp.sum(tc)
             + jnp.float32(B * A_CONST))
    return total / jnp.asarray(norm).astype(jnp.float32)
